# baseline (speedup 1.0000x reference)
import functools

import jax
import jax.numpy as jnp
from jax.experimental import pallas as pl
from jax.experimental.pallas import tpu as pltpu

_IMG = 16
_N = 16384
_REAL = 8649
_TOPN = 1000
_SEL = 1024
_BLK = 128
_NBLK = _SEL // _BLK
_OUT = 256
_THR = 0.8
_EPS = 1e-8


def _iou_block(y1a, x1a, y2a, x2a, aa, y1b, x1b, y2b, x2b, ab):
    yy1 = jnp.maximum(y1a, y1b)
    xx1 = jnp.maximum(x1a, x1b)
    yy2 = jnp.minimum(y2a, y2b)
    xx2 = jnp.minimum(x2a, x2b)
    inter = jnp.maximum(yy2 - yy1, 0.0) * jnp.maximum(xx2 - xx1, 0.0)
    return inter / (aa + ab - inter + _EPS)


def _kernel(d_ref, p_ref, a_ref, out_ref, srtf, srti, scr):
    f32 = jnp.float32
    lane = jax.lax.broadcasted_iota(jnp.int32, (_IMG, _N), 1)

    a0 = a_ref[0]
    a1 = a_ref[1]
    a2 = a_ref[2]
    a3 = a_ref[3]
    ah = a2 - a0
    aw = a3 - a1
    acy = a0 + 0.5 * ah
    acx = a1 + 0.5 * aw
    d0 = d_ref[0] * 0.1
    d1 = d_ref[1] * 0.1
    d2 = d_ref[2] * 0.2
    d3 = d_ref[3] * 0.2
    bh = jnp.exp(d2) * ah
    bw = jnp.exp(d3) * aw
    bcy = d0 * ah + acy
    bcx = d1 * aw + acx
    y1 = bcy - 0.5 * bh
    x1 = bcx - 0.5 * bw
    srtf[0] = p_ref[...]
    srtf[1] = y1
    srtf[2] = x1
    srtf[3] = y1 + bh
    srtf[4] = x1 + bw
    srti[...] = lane

    def sort_pass(_, kj):
        k, j = kj
        s = srtf[0]
        idx = srti[...]
        bit_j = (lane & j) != 0
        nj = -j

        def partner(x):
            return jnp.where(bit_j, pltpu.roll(x, j, 1), pltpu.roll(x, nj, 1))

        ps = partner(s)
        pidx = partner(idx)
        partner_better = (ps > s) | ((ps == s) & (pidx < idx))
        dir_up = (lane & k) == 0
        want_better = jnp.logical_xor(dir_up, bit_j)
        take = ~jnp.logical_xor(partner_better, want_better)
        srtf[0] = jnp.where(take, ps, s)
        srti[...] = jnp.where(take, pidx, idx)
        for c in range(1, 5):
            x = srtf[c]
            srtf[c] = jnp.where(take, partner(x), x)
        done = j == 1
        return (jnp.where(done, k * 2, k), jnp.where(done, k, j // 2))

    jax.lax.fori_loop(0, 105, sort_pass, (jnp.int32(2), jnp.int32(1)),
                      unroll=False)

    lane_s = jax.lax.broadcasted_iota(jnp.int32, (_IMG, _SEL), 1)
    keep0 = (lane_s < _TOPN).astype(f32)
    sy1 = srtf[1][:, :_SEL]
    sx1 = srtf[2][:, :_SEL]
    sy2 = srtf[3][:, :_SEL]
    sx2 = srtf[4][:, :_SEL]
    area = jnp.maximum(sy2 - sy1, 0.0) * jnp.maximum(sx2 - sx1, 0.0)
    ssc = srtf[0][:, :_SEL]
    for t in range(_NBLK):
        sl = slice(t * _BLK, (t + 1) * _BLK)
        scr[0, t] = sy1[:, sl]
        scr[1, t] = sx1[:, sl]
        scr[2, t] = sy2[:, sl]
        scr[3, t] = sx2[:, sl]
        scr[4, t] = ssc[:, sl]
        scr[5, t] = keep0[:, sl]
        scr[6, t] = area[:, sl]

    lane_b = jax.lax.broadcasted_iota(jnp.int32, (_IMG, _BLK), 1)

    def nms_block(t, _):
        by1 = scr[0, t]
        bx1 = scr[1, t]
        by2 = scr[2, t]
        bx2 = scr[3, t]
        ba = scr[6, t]
        kb = scr[5, t]

        def cross(u, kb):
            uy1 = scr[0, u][:, :, None]
            ux1 = scr[1, u][:, :, None]
            uy2 = scr[2, u][:, :, None]
            ux2 = scr[3, u][:, :, None]
            ua = scr[6, u][:, :, None]
            ku = scr[5, u][:, :, None]
            iou = _iou_block(uy1, ux1, uy2, ux2, ua,
                             by1[:, None, :], bx1[:, None, :],
                             by2[:, None, :], bx2[:, None, :], ba[:, None, :])
            sup = jnp.max(((iou > _THR).astype(f32)) * ku, axis=1)
            return kb * (1.0 - sup)

        kb = jax.lax.fori_loop(0, t, cross, kb, unroll=False)

        def inner(i, kb):
            m = (lane_b == i).astype(f32)
            cy1 = jnp.sum(by1 * m, axis=1, keepdims=True)
            cx1 = jnp.sum(bx1 * m, axis=1, keepdims=True)
            cy2 = jnp.sum(by2 * m, axis=1, keepdims=True)
            cx2 = jnp.sum(bx2 * m, axis=1, keepdims=True)
            ca = jnp.sum(ba * m, axis=1, keepdims=True)
            kc = jnp.sum(kb * m, axis=1, keepdims=True)
            iou = _iou_block(cy1, cx1, cy2, cx2, ca, by1, bx1, by2, bx2, ba)
            sup = ((iou > _THR) & (lane_b > i)).astype(f32) * kc
            return kb * (1.0 - sup)

        kb = jax.lax.fori_loop(0, _BLK, inner, kb, unroll=False)
        scr[5, t] = kb
        return 0

    jax.lax.fori_loop(0, _NBLK, nms_block, 0, unroll=False)

    keep = jnp.concatenate([scr[5, t] for t in range(_NBLK)], axis=1)
    csum = keep
    for sh in (1, 2, 4, 8, 16, 32, 64, 128, 256, 512):
        csum = csum + jnp.where(lane_s >= sh, pltpu.roll(csum, sh, 1), 0.0)
    posm = jnp.where(keep > 0.5, csum - 1.0, -1.0)
    nk = jnp.sum(keep, axis=1, keepdims=True)

    io_out = jax.lax.broadcasted_iota(jnp.int32, (_IMG, _BLK, _OUT), 2)
    io_out = io_out.astype(f32)
    acc = jnp.zeros((_IMG, 8, _OUT), f32)
    zrow = jnp.zeros((_IMG, 3, _BLK), f32)
    for t in range(_NBLK):
        sl = slice(t * _BLK, (t + 1) * _BLK)
        oh = (posm[:, sl, None] == io_out).astype(f32)
        v = jnp.concatenate(
            [scr[c, t][:, None, :] for c in range(5)] + [zrow], axis=1)
        acc = acc + jax.lax.dot_general(
            v, oh, dimension_numbers=(((2,), (1,)), ((0,), (0,))),
            preferred_element_type=f32)

    row = jax.lax.broadcasted_iota(jnp.int32, (_IMG, 8, _OUT), 1)
    clipped = jnp.clip(acc, 0.0, 1.0)
    outv = jnp.where(row < 4, clipped, acc)
    valid = (jax.lax.broadcasted_iota(jnp.int32, (_IMG, _OUT), 1).astype(f32)
             < nk)[:, None, :]
    out_ref[...] = jnp.where(valid, outv, 0.0)


@jax.jit
def kernel(rpn_bbox_deltas, rpn_probs, gt_labels, anchors):
    del gt_labels
    b = rpn_bbox_deltas.shape[0]
    f32 = jnp.float32
    d = rpn_bbox_deltas.reshape(b, _REAL, 4).astype(f32)
    d = jnp.moveaxis(d, 2, 0)
    d = jnp.pad(d, ((0, 0), (0, 0), (0, _N - _REAL)))
    p = rpn_probs.reshape(b, _REAL).astype(f32)
    p = jnp.pad(p, ((0, 0), (0, _N - _REAL)), constant_values=-1.0)
    a = jnp.moveaxis(anchors.astype(f32), 1, 0)[:, None, :]
    a = jnp.pad(a, ((0, 0), (0, 0), (0, _N - _REAL)))

    out = pl.pallas_call(
        _kernel,
        out_shape=jax.ShapeDtypeStruct((b, 8, _OUT), f32),
        in_specs=[
            pl.BlockSpec(memory_space=pltpu.VMEM),
            pl.BlockSpec(memory_space=pltpu.VMEM),
            pl.BlockSpec(memory_space=pltpu.VMEM),
        ],
        out_specs=pl.BlockSpec(memory_space=pltpu.VMEM),
        scratch_shapes=[
            pltpu.VMEM((5, _IMG, _N), f32),
            pltpu.VMEM((_IMG, _N), jnp.int32),
            pltpu.VMEM((7, _NBLK, _IMG, _BLK), f32),
        ],
        compiler_params=pltpu.CompilerParams(
            vmem_limit_bytes=48 * 1024 * 1024,
        ),
    )(d, p, a)

    roi_bboxes = jnp.transpose(out[:, :4, :], (0, 2, 1))
    roi_scores = out[:, 4, :]
    return roi_bboxes, roi_scores
